# baseline (speedup 1.0000x reference)
import functools

import jax
import jax.numpy as jnp
from jax import lax
from jax.experimental import pallas as pl
from jax.experimental.pallas import tpu as pltpu
from jax.experimental.pallas import tpu_sc as plsc

_NUM_CORES = 2


def kernel(relative_position_bias_table, seq_len):
    table = relative_position_bias_table
    h = table.shape[1]
    n = (table.shape[0] + 1) // 2
    width = 2 * n
    r = jnp.flip(table, axis=0).T
    r = jnp.pad(r, ((0, 0), (0, 9)))
    rcop = jnp.stack([r[:, f:f + width] for f in reversed(range(8))], axis=1)
    rcop = rcop.reshape(h, 8 * width)

    rows_per_w = n // 2
    groups = rows_per_w // 8
    mesh = plsc.VectorSubcoreMesh(core_axis_name="c", subcore_axis_name="s")

    @functools.partial(
        pl.kernel, mesh=mesh,
        out_type=jax.ShapeDtypeStruct((h * n * n,), table.dtype),
        scratch_types=[pltpu.VMEM((8 * width,), jnp.float32),
                       pltpu.SemaphoreType.DMA],
    )
    def sck(rcop_hbm, out_hbm, rloc, sem):
        wid = lax.axis_index("s") * _NUM_CORES + lax.axis_index("c")
        hd = wid // 2
        i_base = (wid % 2) * rows_per_w
        pltpu.sync_copy(rcop_hbm.at[hd], rloc)

        def row_copy(g, k):
            i0 = i_base + 8 * g
            return pltpu.make_async_copy(
                rloc.at[pl.ds(k * width + n - 8 - i0, n)],
                out_hbm.at[pl.ds((hd * n + i0 + k) * n, n)],
                sem)

        @pl.loop(0, groups)
        def _fire(g):
            for k in range(8):
                row_copy(g, k).start()

        @pl.loop(0, groups)
        def _drain(g):
            for k in range(8):
                row_copy(g, k).wait()

    return sck(rcop).reshape(1, h, n, n)


# speedup vs baseline: 3.5130x; 3.5130x over previous
import jax
import jax.numpy as jnp
from jax.experimental import pallas as pl
from jax.experimental.pallas import tpu as pltpu


def _toeplitz_body(r_ref, o_ref, scratch_ref, *sems):
    h = r_ref.shape[0]
    two_n = r_ref.shape[1]
    n = two_n // 2

    rows = r_ref[...]
    for d in range(128):
        shift = 127 - d
        scratch_ref[d, :, :] = pltpu.roll(rows, (two_n - shift) % two_n, 1)

    copies = []
    for hh in range(h):
        for c in range(n // 128):
            b0 = n - 128 - 128 * c
            copies.append(pltpu.make_async_copy(
                scratch_ref.at[:, hh, pl.ds(b0, n)],
                o_ref.at[0, hh, pl.ds(128 * c, 128), :],
                sems[(hh * (n // 128) + c) % len(sems)]))
    for cp in copies:
        cp.start()
    for cp in copies:
        cp.wait()


def kernel(relative_position_bias_table, seq_len):
    table = relative_position_bias_table
    h = table.shape[1]
    n = (table.shape[0] + 1) // 2
    r = jnp.flip(table, axis=0).T
    r = jnp.pad(r, ((0, 0), (0, 1)))

    out = pl.pallas_call(
        _toeplitz_body,
        in_specs=[pl.BlockSpec(memory_space=pltpu.MemorySpace.VMEM)],
        out_specs=pl.BlockSpec(memory_space=pltpu.MemorySpace.HBM),
        out_shape=jax.ShapeDtypeStruct((1, h, n, n), table.dtype),
        scratch_shapes=[pltpu.VMEM((128, h, 2 * n), table.dtype)] +
                       [pltpu.SemaphoreType.DMA] * 4,
    )(r)
    return out
